# baseline (device time: 38934 ns/iter reference)
import jax
import jax.numpy as jnp
from jax import lax
from jax.experimental import pallas as pl
from jax.experimental.pallas import tpu as pltpu

N_DEV = 32


def kernel(x, w_mat):
    m, _ = x.shape
    _, n = w_mat.shape
    m_blk = m // N_DEV

    def body(x_ref, w_ref, out_ref, partial_ref, recv_ref, send_sems, recv_sems):
        my = lax.axis_index("i")

        for i in range(N_DEV):
            blk = jnp.dot(
                x_ref[i * m_blk : (i + 1) * m_blk, :],
                w_ref[:, :],
                preferred_element_type=jnp.float32,
            )
            partial_ref[i, :, :] = blk.astype(jnp.bfloat16)

        rdmas = []
        for j in range(1, N_DEV):
            dst = lax.rem(my + j, N_DEV)
            rdma = pltpu.make_async_remote_copy(
                src_ref=partial_ref.at[dst],
                dst_ref=recv_ref.at[j - 1],
                send_sem=send_sems.at[j - 1],
                recv_sem=recv_sems.at[j - 1],
                device_id=(dst,),
                device_id_type=pl.DeviceIdType.MESH,
            )
            rdma.start()
            rdmas.append(rdma)

        acc = jnp.dot(
            x_ref[pl.ds(my * m_blk, m_blk), :],
            w_ref[:, :],
            preferred_element_type=jnp.float32,
        )
        for j in range(1, N_DEV):
            rdmas[j - 1].wait_recv()
            acc = acc + recv_ref[j - 1, :, :].astype(jnp.float32)

        out_ref[:, :] = acc * jax.nn.sigmoid(acc)

        for r in rdmas:
            r.wait_send()

    return pl.pallas_call(
        body,
        out_shape=jax.ShapeDtypeStruct((m_blk, n), jnp.float32),
        in_specs=[
            pl.BlockSpec(memory_space=pltpu.VMEM),
            pl.BlockSpec(memory_space=pltpu.VMEM),
        ],
        out_specs=pl.BlockSpec(memory_space=pltpu.VMEM),
        scratch_shapes=[
            pltpu.VMEM((N_DEV, m_blk, n), jnp.bfloat16),
            pltpu.VMEM((N_DEV - 1, m_blk, n), jnp.bfloat16),
            pltpu.SemaphoreType.DMA((N_DEV - 1,)),
            pltpu.SemaphoreType.DMA((N_DEV - 1,)),
        ],
    )(x, w_mat)


# device time: 37752 ns/iter; 1.0313x vs baseline; 1.0313x over previous
import functools

import jax
import jax.numpy as jnp
from jax import lax
from jax.experimental import pallas as pl
from jax.experimental.pallas import tpu as pltpu

N_DEV = 32
N_PLANE = 16


def _coords(i):
    z, r = divmod(i, 8)
    y = r // 2
    x = (r % 2) if y % 2 == 0 else 1 - (r % 2)
    return x, y, z


_X = [_coords(i)[0] for i in range(N_DEV)]
_K = [_coords(i)[2] * 4 + _coords(i)[1] for i in range(N_DEV)]


def kernel(x, w_mat):
    m, _ = x.shape
    _, n = w_mat.shape
    m_blk = m // N_DEV

    def body(
        x_ref,
        w_ref,
        out_ref,
        part_ref,
        xrecv_ref,
        comb_ref,
        prec_ref,
        xsend_sems,
        xrecv_sems,
        psend_sems,
        precv_sems,
    ):
        my = lax.axis_index("i")
        my_x = (my + (my >> 1)) & 1
        my_k = (my >> 3) * 4 + ((my & 7) >> 1)
        partner = my ^ 1
        other = 1 - my_x

        def plane_id(k):
            z = k >> 2
            y = k & 3
            xterm = jnp.where((y & 1) == 0, my_x, 1 - my_x)
            return z * 8 + 2 * y + xterm

        for d in range(N_DEV):
            blk = jnp.dot(
                x_ref[d * m_blk : (d + 1) * m_blk, :],
                w_ref[:, :],
                preferred_element_type=jnp.float32,
            )
            part_ref[_X[d] * N_PLANE + _K[d], :, :] = blk.astype(jnp.bfloat16)

        xr = []
        for o in range(N_PLANE):
            src_k = lax.rem(my_k + o, N_PLANE)
            r = pltpu.make_async_remote_copy(
                src_ref=part_ref.at[other * N_PLANE + src_k],
                dst_ref=xrecv_ref.at[o],
                send_sem=xsend_sems.at[o],
                recv_sem=xrecv_sems.at[o],
                device_id=(partner,),
                device_id_type=pl.DeviceIdType.MESH,
            )
            r.start()
            xr.append(r)

        rdmas = []
        for o in range(1, N_PLANE):
            dst_k = lax.rem(my_k + o, N_PLANE)
            xr[o].wait_recv()
            comb = part_ref[my_x * N_PLANE + dst_k, :, :].astype(jnp.float32) + xrecv_ref[
                o, :, :
            ].astype(jnp.float32)
            comb_ref[o, :, :] = comb.astype(jnp.bfloat16)
            r = pltpu.make_async_remote_copy(
                src_ref=comb_ref.at[o],
                dst_ref=prec_ref.at[o - 1],
                send_sem=psend_sems.at[o - 1],
                recv_sem=precv_sems.at[o - 1],
                device_id=(plane_id(dst_k),),
                device_id_type=pl.DeviceIdType.MESH,
            )
            r.start()
            rdmas.append(r)

        xr[0].wait_recv()
        acc = part_ref[my_x * N_PLANE + my_k, :, :].astype(jnp.float32) + xrecv_ref[
            0, :, :
        ].astype(jnp.float32)
        for o in range(1, N_PLANE):
            rdmas[o - 1].wait_recv()
            acc = acc + prec_ref[o - 1, :, :].astype(jnp.float32)

        out_ref[:, :] = acc * jax.nn.sigmoid(acc)

        for r in xr:
            r.wait_send()
        for r in rdmas:
            r.wait_send()

        @functools.partial(pl.run_scoped, done_sem=pltpu.SemaphoreType.REGULAR)
        def _(done_sem):
            pl.semaphore_signal(
                done_sem,
                inc=1,
                device_id=(partner,),
                device_id_type=pl.DeviceIdType.MESH,
            )
            for o in range(1, N_PLANE):
                dst_k = lax.rem(my_k + o, N_PLANE)
                pl.semaphore_signal(
                    done_sem,
                    inc=1,
                    device_id=(plane_id(dst_k),),
                    device_id_type=pl.DeviceIdType.MESH,
                )
            pl.semaphore_wait(done_sem, N_PLANE)

    return pl.pallas_call(
        body,
        out_shape=jax.ShapeDtypeStruct((m_blk, n), jnp.float32),
        in_specs=[
            pl.BlockSpec(memory_space=pltpu.VMEM),
            pl.BlockSpec(memory_space=pltpu.VMEM),
        ],
        out_specs=pl.BlockSpec(memory_space=pltpu.VMEM),
        scratch_shapes=[
            pltpu.VMEM((2 * N_PLANE, m_blk, n), jnp.bfloat16),
            pltpu.VMEM((N_PLANE, m_blk, n), jnp.bfloat16),
            pltpu.VMEM((N_PLANE, m_blk, n), jnp.bfloat16),
            pltpu.VMEM((N_PLANE - 1, m_blk, n), jnp.bfloat16),
            pltpu.SemaphoreType.DMA((N_PLANE,)),
            pltpu.SemaphoreType.DMA((N_PLANE,)),
            pltpu.SemaphoreType.DMA((N_PLANE - 1,)),
            pltpu.SemaphoreType.DMA((N_PLANE - 1,)),
        ],
    )(x, w_mat)


# device time: 26969 ns/iter; 1.4437x vs baseline; 1.3998x over previous
import jax
import jax.numpy as jnp
from jax import lax
from jax.experimental import pallas as pl
from jax.experimental.pallas import tpu as pltpu

N_DEV = 32
N_PLANE = 16


def _coords(i):
    z, r = divmod(i, 8)
    y = r // 2
    x = (r % 2) if y % 2 == 0 else 1 - (r % 2)
    return x, y, z


_X = [_coords(i)[0] for i in range(N_DEV)]
_K = [_coords(i)[2] * 4 + _coords(i)[1] for i in range(N_DEV)]


def kernel(x, w_mat):
    m, _ = x.shape
    _, n = w_mat.shape
    m_blk = m // N_DEV

    def body(
        x_ref,
        w_ref,
        out_ref,
        part_ref,
        xrecv_ref,
        comb_ref,
        prec_ref,
        xsend_sems,
        xrecv_sems,
        psend_sems,
        precv_sems,
    ):
        my = lax.axis_index("i")
        my_x = (my + (my >> 1)) & 1
        my_k = (my >> 3) * 4 + ((my & 7) >> 1)
        partner = my ^ 1
        other = 1 - my_x

        def plane_id(k):
            z = k >> 2
            y = k & 3
            xterm = jnp.where((y & 1) == 0, my_x, 1 - my_x)
            return z * 8 + 2 * y + xterm

        barrier_sem = pltpu.get_barrier_semaphore()
        pl.semaphore_signal(
            barrier_sem,
            inc=1,
            device_id=(partner,),
            device_id_type=pl.DeviceIdType.MESH,
        )
        for o in range(1, N_PLANE):
            dst_k = lax.rem(my_k + o, N_PLANE)
            pl.semaphore_signal(
                barrier_sem,
                inc=1,
                device_id=(plane_id(dst_k),),
                device_id_type=pl.DeviceIdType.MESH,
            )
        pl.semaphore_wait(barrier_sem, N_PLANE)

        for d in range(N_DEV):
            blk = jnp.dot(
                x_ref[d * m_blk : (d + 1) * m_blk, :],
                w_ref[:, :],
                preferred_element_type=jnp.float32,
            )
            part_ref[_X[d] * N_PLANE + _K[d], :, :] = blk.astype(jnp.bfloat16)

        xr = []
        for o in range(N_PLANE):
            src_k = lax.rem(my_k + o, N_PLANE)
            r = pltpu.make_async_remote_copy(
                src_ref=part_ref.at[other * N_PLANE + src_k],
                dst_ref=xrecv_ref.at[o],
                send_sem=xsend_sems.at[o],
                recv_sem=xrecv_sems.at[o],
                device_id=(partner,),
                device_id_type=pl.DeviceIdType.MESH,
            )
            r.start()
            xr.append(r)

        rdmas = []
        for o in range(1, N_PLANE):
            dst_k = lax.rem(my_k + o, N_PLANE)
            xr[o].wait_recv()
            comb = part_ref[my_x * N_PLANE + dst_k, :, :].astype(jnp.float32) + xrecv_ref[
                o, :, :
            ].astype(jnp.float32)
            comb_ref[o, :, :] = comb.astype(jnp.bfloat16)
            r = pltpu.make_async_remote_copy(
                src_ref=comb_ref.at[o],
                dst_ref=prec_ref.at[o - 1],
                send_sem=psend_sems.at[o - 1],
                recv_sem=precv_sems.at[o - 1],
                device_id=(plane_id(dst_k),),
                device_id_type=pl.DeviceIdType.MESH,
            )
            r.start()
            rdmas.append(r)

        xr[0].wait_recv()
        acc = part_ref[my_x * N_PLANE + my_k, :, :].astype(jnp.float32) + xrecv_ref[
            0, :, :
        ].astype(jnp.float32)
        for o in range(1, N_PLANE):
            rdmas[o - 1].wait_recv()
            acc = acc + prec_ref[o - 1, :, :].astype(jnp.float32)

        out_ref[:, :] = acc * jax.nn.sigmoid(acc)

        for r in xr:
            r.wait_send()
        for r in rdmas:
            r.wait_send()

    return pl.pallas_call(
        body,
        out_shape=jax.ShapeDtypeStruct((m_blk, n), jnp.float32),
        in_specs=[
            pl.BlockSpec(memory_space=pltpu.VMEM),
            pl.BlockSpec(memory_space=pltpu.VMEM),
        ],
        out_specs=pl.BlockSpec(memory_space=pltpu.VMEM),
        scratch_shapes=[
            pltpu.VMEM((2 * N_PLANE, m_blk, n), jnp.bfloat16),
            pltpu.VMEM((N_PLANE, m_blk, n), jnp.bfloat16),
            pltpu.VMEM((N_PLANE, m_blk, n), jnp.bfloat16),
            pltpu.VMEM((N_PLANE - 1, m_blk, n), jnp.bfloat16),
            pltpu.SemaphoreType.DMA((N_PLANE,)),
            pltpu.SemaphoreType.DMA((N_PLANE,)),
            pltpu.SemaphoreType.DMA((N_PLANE - 1,)),
            pltpu.SemaphoreType.DMA((N_PLANE - 1,)),
        ],
        compiler_params=pltpu.CompilerParams(collective_id=0),
    )(x, w_mat)
